# Initial kernel scaffold; baseline (speedup 1.0000x reference)
import functools

import jax
import jax.numpy as jnp
from jax import lax
from jax.experimental import pallas as pl
from jax.experimental.pallas import tpu as pltpu
from jax.experimental.pallas import tpu_sc as plsc

N_NODES = 10000
NUM_EDGES = 10000
N_INC = 320000
C = 128

NC = 2
NS = 16
NW = NC * NS
PER_W = N_INC // NW
CHUNK = 80
N_CHUNKS = PER_W // CHUNK
SEG = N_NODES
ROWS_PER_TILE = SEG // NS


def _mm_body(x_ref, w_ref, o_ref):
    o_ref[...] = jnp.dot(x_ref[...], w_ref[...],
                         preferred_element_type=jnp.float32)


def _matmul(x, W):
    return pl.pallas_call(
        _mm_body,
        grid=(10,),
        in_specs=[
            pl.BlockSpec((N_NODES // 10, C), lambda i: (i, 0)),
            pl.BlockSpec((C, C), lambda i: (0, 0)),
        ],
        out_specs=pl.BlockSpec((N_NODES // 10, C), lambda i: (i, 0)),
        out_shape=jax.ShapeDtypeStruct((N_NODES, C), jnp.float32),
    )(x, W)


def _make_stage(with_counts):
    mesh = plsc.VectorSubcoreMesh(core_axis_name="c", subcore_axis_name="s")
    out_type = [jax.ShapeDtypeStruct((NC, SEG, C), jnp.float32)]
    scratch = [
        pltpu.VMEM((CHUNK,), jnp.int32),
        pltpu.VMEM((CHUNK,), jnp.int32),
        pltpu.VMEM((CHUNK, C), jnp.float32),
        pltpu.VMEM((ROWS_PER_TILE, C), jnp.float32),
        pltpu.VMEM_SHARED((SEG, C), jnp.float32),
        pltpu.SemaphoreType.DMA,
    ]
    if with_counts:
        out_type += [
            jax.ShapeDtypeStruct((NC, NUM_EDGES, 16), jnp.float32),
            jax.ShapeDtypeStruct((NC, N_NODES, 16), jnp.float32),
        ]
        scratch += [
            pltpu.VMEM((CHUNK, 16), jnp.float32),
            pltpu.VMEM((ROWS_PER_TILE, 16), jnp.float32),
            pltpu.VMEM_SHARED((NUM_EDGES, 16), jnp.float32),
            pltpu.VMEM_SHARED((N_NODES, 16), jnp.float32),
        ]

    def body(src_hbm, gidx_hbm, sidx_hbm, out_hbm, *rest):
        if with_counts:
            (cnt_hbm, dcnt_hbm,
             gidx_v, sidx_v, rows_v, wb_v, acc_sh, sem,
             ones_v, cwb_v, cnt_sh, dcnt_sh) = rest
        else:
            (gidx_v, sidx_v, rows_v, wb_v, acc_sh, sem) = rest
        cid = lax.axis_index("c")
        sid = lax.axis_index("s")
        wid = cid * NS + sid
        base = wid * PER_W
        stripe = sid * ROWS_PER_TILE

        z16 = jnp.zeros((16,), jnp.float32)

        def zero_wb(i, _):
            for cblk in range(C // 16):
                wb_v[i, pl.ds(cblk * 16, 16)] = z16
            return 0

        lax.fori_loop(0, ROWS_PER_TILE, zero_wb, 0)
        pltpu.sync_copy(wb_v, acc_sh.at[pl.ds(stripe, ROWS_PER_TILE)])

        if with_counts:
            o16 = jnp.ones((16,), jnp.float32)

            def zero_cwb(i, _):
                cwb_v[i, :] = z16
                return 0

            lax.fori_loop(0, ROWS_PER_TILE, zero_cwb, 0)
            pltpu.sync_copy(cwb_v, cnt_sh.at[pl.ds(stripe, ROWS_PER_TILE)])
            pltpu.sync_copy(cwb_v, dcnt_sh.at[pl.ds(stripe, ROWS_PER_TILE)])

            def fill_ones(i, _):
                ones_v[i, :] = o16
                return 0

            lax.fori_loop(0, CHUNK, fill_ones, 0)

        plsc.subcore_barrier()

        def step(j, _):
            off = base + j * CHUNK
            pltpu.sync_copy(gidx_hbm.at[pl.ds(off, CHUNK)], gidx_v)
            pltpu.sync_copy(sidx_hbm.at[pl.ds(off, CHUNK)], sidx_v)
            pltpu.async_copy(src_hbm.at[gidx_v], rows_v, sem).wait()
            pltpu.sync_copy(rows_v, acc_sh.at[sidx_v], add=True)
            if with_counts:
                pltpu.sync_copy(ones_v, cnt_sh.at[sidx_v], add=True)
                pltpu.sync_copy(ones_v, dcnt_sh.at[gidx_v], add=True)
            return 0

        lax.fori_loop(0, N_CHUNKS, step, 0)
        plsc.subcore_barrier()

        pltpu.sync_copy(acc_sh.at[pl.ds(stripe, ROWS_PER_TILE)], wb_v)
        pltpu.sync_copy(wb_v, out_hbm.at[cid, pl.ds(stripe, ROWS_PER_TILE)])
        if with_counts:
            pltpu.sync_copy(cnt_sh.at[pl.ds(stripe, ROWS_PER_TILE)], cwb_v)
            pltpu.sync_copy(cwb_v, cnt_hbm.at[cid, pl.ds(stripe, ROWS_PER_TILE)])
            pltpu.sync_copy(dcnt_sh.at[pl.ds(stripe, ROWS_PER_TILE)], cwb_v)
            pltpu.sync_copy(cwb_v, dcnt_hbm.at[cid, pl.ds(stripe, ROWS_PER_TILE)])

    return functools.partial(
        pl.kernel, mesh=mesh, out_type=out_type, scratch_types=scratch
    )(body)


_stage_counts = _make_stage(with_counts=True)
_stage_plain = _make_stage(with_counts=False)


def _comb_body(part_ref, cnt_ref, o_ref):
    s = part_ref[0] + part_ref[1]
    c = cnt_ref[0][:, 0:1] + cnt_ref[1][:, 0:1]
    inv = jnp.where(c > 0.0, 1.0 / c, 0.0)
    o_ref[...] = s * inv


def _combine(part, cnt):
    blk = SEG // 10
    return pl.pallas_call(
        _comb_body,
        grid=(10,),
        in_specs=[
            pl.BlockSpec((NC, blk, C), lambda i: (0, i, 0)),
            pl.BlockSpec((NC, blk, 16), lambda i: (0, i, 0)),
        ],
        out_specs=pl.BlockSpec((blk, C), lambda i: (i, 0)),
        out_shape=jax.ShapeDtypeStruct((SEG, C), jnp.float32),
    )(part, cnt)


def kernel(x, hyperedge_index, W):
    he = hyperedge_index.astype(jnp.int32)
    row = he[0]
    col = he[1]
    xw = _matmul(x, W)
    e_part, cnt, dcnt = _stage_counts(xw, row, col)
    edge_feat = _combine(e_part, cnt)
    n_part = _stage_plain(edge_feat, col, row)
    out = _combine(n_part, dcnt)
    return out


# speedup vs baseline: 15.0681x; 15.0681x over previous
import functools

import jax
import jax.numpy as jnp
from jax import lax
from jax.experimental import pallas as pl
from jax.experimental.pallas import tpu as pltpu
from jax.experimental.pallas import tpu_sc as plsc

N_NODES = 10000
NUM_EDGES = 10000
N_INC = 320000
C = 128

NC = 2
NS = 16
NW = NC * NS
PER_W = N_INC // NW
CHUNK = 80
N_CHUNKS = PER_W // CHUNK
SEG = N_NODES
SEG_CHUNKS = SEG // CHUNK


def _mm_body(x_ref, w_ref, o_ref):
    o_ref[...] = jnp.dot(x_ref[...], w_ref[...],
                         preferred_element_type=jnp.float32)


def _matmul(x, W):
    return pl.pallas_call(
        _mm_body,
        grid=(10,),
        in_specs=[
            pl.BlockSpec((N_NODES // 10, C), lambda i: (i, 0)),
            pl.BlockSpec((C, C), lambda i: (0, 0)),
        ],
        out_specs=pl.BlockSpec((N_NODES // 10, C), lambda i: (i, 0)),
        out_shape=jax.ShapeDtypeStruct((N_NODES, C), jnp.float32),
    )(x, W)


def _make_stage(with_counts):
    mesh = plsc.VectorSubcoreMesh(core_axis_name="c", subcore_axis_name="s")
    out_type = [jax.ShapeDtypeStruct((NC, SEG, C), jnp.float32)]
    scratch = [
        pltpu.VMEM((CHUNK,), jnp.int32),
        pltpu.VMEM((CHUNK,), jnp.int32),
        pltpu.VMEM((CHUNK, C), jnp.float32),
        pltpu.VMEM_SHARED((SEG, C), jnp.float32),
        pltpu.SemaphoreType.DMA,
    ]
    if with_counts:
        out_type += [
            jax.ShapeDtypeStruct((NC * NUM_EDGES,), jnp.float32),
            jax.ShapeDtypeStruct((NC * N_NODES,), jnp.float32),
        ]
        scratch += [
            pltpu.VMEM((CHUNK,), jnp.float32),
            pltpu.VMEM((CHUNK,), jnp.float32),
            pltpu.VMEM_SHARED((NUM_EDGES,), jnp.float32),
            pltpu.VMEM_SHARED((N_NODES,), jnp.float32),
        ]

    def body(src_hbm, gidx_hbm, sidx_hbm, out_hbm, *rest):
        if with_counts:
            (cnt_hbm, dcnt_hbm,
             gidx_v, sidx_v, rows_v, acc_sh, sem,
             ones_v, stg_v, cnt_sh, dcnt_sh) = rest
        else:
            (gidx_v, sidx_v, rows_v, acc_sh, sem) = rest
        cid = lax.axis_index("c")
        sid = lax.axis_index("s")
        wid = cid * NS + sid
        base = wid * PER_W
        nj = (SEG_CHUNKS + NS - 1) // NS

        def chunk_id(i):
            return jnp.minimum(sid + i * NS, SEG_CHUNKS - 1)

        z16 = jnp.zeros((16,), jnp.float32)

        def zero_rows(i, _):
            for cblk in range(C // 16):
                rows_v[i, pl.ds(cblk * 16, 16)] = z16
            return 0

        lax.fori_loop(0, CHUNK, zero_rows, 0)

        def zinit(i, _):
            j = chunk_id(i)
            pltpu.sync_copy(rows_v, acc_sh.at[pl.ds(j * CHUNK, CHUNK)])
            return 0

        lax.fori_loop(0, nj, zinit, 0)

        if with_counts:
            o16 = jnp.ones((16,), jnp.float32)
            for k in range(CHUNK // 16):
                ones_v[pl.ds(k * 16, 16)] = o16
                stg_v[pl.ds(k * 16, 16)] = z16

            def zinit_cnt(i, _):
                j = chunk_id(i)
                pltpu.sync_copy(stg_v, cnt_sh.at[pl.ds(j * CHUNK, CHUNK)])
                pltpu.sync_copy(stg_v, dcnt_sh.at[pl.ds(j * CHUNK, CHUNK)])
                return 0

            lax.fori_loop(0, nj, zinit_cnt, 0)

        plsc.subcore_barrier()

        def step(j, _):
            off = base + j * CHUNK
            pltpu.sync_copy(gidx_hbm.at[pl.ds(off, CHUNK)], gidx_v)
            pltpu.sync_copy(sidx_hbm.at[pl.ds(off, CHUNK)], sidx_v)
            pltpu.async_copy(src_hbm.at[gidx_v], rows_v, sem).wait()
            pltpu.sync_copy(rows_v, acc_sh.at[sidx_v], add=True)
            if with_counts:
                pltpu.sync_copy(ones_v, cnt_sh.at[sidx_v], add=True)
                pltpu.sync_copy(ones_v, dcnt_sh.at[gidx_v], add=True)
            return 0

        lax.fori_loop(0, N_CHUNKS, step, 0)
        plsc.subcore_barrier()

        def wback(i, _):
            j = chunk_id(i)
            pltpu.sync_copy(acc_sh.at[pl.ds(j * CHUNK, CHUNK)], rows_v)
            pltpu.sync_copy(rows_v, out_hbm.at[cid, pl.ds(j * CHUNK, CHUNK)])
            return 0

        lax.fori_loop(0, nj, wback, 0)
        if with_counts:
            def wback_cnt(i, _):
                j = chunk_id(i)
                pltpu.sync_copy(cnt_sh.at[pl.ds(j * CHUNK, CHUNK)], stg_v)
                pltpu.sync_copy(
                    stg_v, cnt_hbm.at[pl.ds(cid * NUM_EDGES + j * CHUNK, CHUNK)])
                pltpu.sync_copy(dcnt_sh.at[pl.ds(j * CHUNK, CHUNK)], ones_v)
                pltpu.sync_copy(
                    ones_v, dcnt_hbm.at[pl.ds(cid * N_NODES + j * CHUNK, CHUNK)])
                return 0

            lax.fori_loop(0, nj, wback_cnt, 0)

    return functools.partial(
        pl.kernel, mesh=mesh, out_type=out_type, scratch_types=scratch
    )(body)


_stage_counts = _make_stage(with_counts=True)
_stage_plain = _make_stage(with_counts=False)


def _comb_body(part_ref, cnt_ref, o_ref):
    s = part_ref[0] + part_ref[1]
    c = (cnt_ref[0] + cnt_ref[1])[:, None]
    inv = jnp.where(c > 0.0, 1.0 / c, 0.0)
    o_ref[...] = s * inv


def _combine(part, cnt):
    return pl.pallas_call(
        _comb_body,
        out_shape=jax.ShapeDtypeStruct((SEG, C), jnp.float32),
    )(part, cnt)


def kernel(x, hyperedge_index, W):
    he = hyperedge_index.astype(jnp.int32)
    row = he[0]
    col = he[1]
    xw = _matmul(x, W)
    e_part, cnt, dcnt = _stage_counts(xw, row, col)
    edge_feat = _combine(e_part, cnt.reshape(NC, NUM_EDGES))
    (n_part,) = _stage_plain(edge_feat, col, row)
    out = _combine(n_part, dcnt.reshape(NC, N_NODES))
    return out


# speedup vs baseline: 28.3879x; 1.8840x over previous
import functools

import jax
import jax.numpy as jnp
from jax import lax
from jax.experimental import pallas as pl
from jax.experimental.pallas import tpu as pltpu
from jax.experimental.pallas import tpu_sc as plsc

N_NODES = 10000
NUM_EDGES = 10000
N_INC = 320000
C = 128

NC = 2
NS = 16
NW = NC * NS
PER_W = N_INC // NW
CHUNK = 80
N_CHUNKS = PER_W // CHUNK
SEG = N_NODES
SEG_CHUNKS = SEG // CHUNK
STEADY0 = 3
STEADY_ITERS = (N_CHUNKS - STEADY0 - 2) // 3


def _mm_body(x_ref, w_ref, o_ref):
    o_ref[...] = jnp.dot(x_ref[...], w_ref[...],
                         preferred_element_type=jnp.float32)


def _matmul(x, W):
    return pl.pallas_call(
        _mm_body,
        grid=(10,),
        in_specs=[
            pl.BlockSpec((N_NODES // 10, C), lambda i: (i, 0)),
            pl.BlockSpec((C, C), lambda i: (0, 0)),
        ],
        out_specs=pl.BlockSpec((N_NODES // 10, C), lambda i: (i, 0)),
        out_shape=jax.ShapeDtypeStruct((N_NODES, C), jnp.float32),
    )(x, W)


def _make_stage(with_counts):
    mesh = plsc.VectorSubcoreMesh(core_axis_name="c", subcore_axis_name="s")
    out_type = [jax.ShapeDtypeStruct((NC, SEG, C), jnp.float32)]
    scratch = [
        pltpu.VMEM((CHUNK,), jnp.int32), pltpu.VMEM((CHUNK,), jnp.int32),
        pltpu.VMEM((CHUNK,), jnp.int32),
        pltpu.VMEM((CHUNK,), jnp.int32), pltpu.VMEM((CHUNK,), jnp.int32),
        pltpu.VMEM((CHUNK,), jnp.int32),
        pltpu.VMEM((CHUNK, C), jnp.float32), pltpu.VMEM((CHUNK, C), jnp.float32),
        pltpu.VMEM((CHUNK, C), jnp.float32),
        pltpu.VMEM_SHARED((SEG, C), jnp.float32),
        pltpu.SemaphoreType.DMA, pltpu.SemaphoreType.DMA,
        pltpu.SemaphoreType.DMA,
        pltpu.SemaphoreType.DMA, pltpu.SemaphoreType.DMA,
        pltpu.SemaphoreType.DMA,
        pltpu.SemaphoreType.DMA,
    ]
    if with_counts:
        out_type += [
            jax.ShapeDtypeStruct((NC * NUM_EDGES,), jnp.float32),
            jax.ShapeDtypeStruct((NC * N_NODES,), jnp.float32),
        ]
        scratch += [
            pltpu.SemaphoreType.DMA,
            pltpu.VMEM((CHUNK,), jnp.float32),
            pltpu.VMEM((CHUNK,), jnp.float32),
            pltpu.VMEM_SHARED((NUM_EDGES,), jnp.float32),
            pltpu.VMEM_SHARED((N_NODES,), jnp.float32),
        ]

    def body(src_hbm, gidx_hbm, sidx_hbm, out_hbm, *rest):
        if with_counts:
            (cnt_hbm, dcnt_hbm,
             gi0, gi1, gi2, si0, si1, si2, r0, r1, r2, acc_sh,
             semi0, semi1, semi2, semg0, semg1, semg2, sem_s,
             sem_c, ones_v, stg_v, cnt_sh, dcnt_sh) = rest
        else:
            (gi0, gi1, gi2, si0, si1, si2, r0, r1, r2, acc_sh,
             semi0, semi1, semi2, semg0, semg1, semg2, sem_s) = rest
        gi = (gi0, gi1, gi2)
        si = (si0, si1, si2)
        r = (r0, r1, r2)
        semi = (semi0, semi1, semi2)
        semg = (semg0, semg1, semg2)

        cid = lax.axis_index("c")
        sid = lax.axis_index("s")
        wid = cid * NS + sid
        base = wid * PER_W
        nj = (SEG_CHUNKS + NS - 1) // NS

        def chunk_id(i):
            return jnp.minimum(sid + i * NS, SEG_CHUNKS - 1)

        def issue_idx(s, j):
            off = base + j * CHUNK
            pltpu.async_copy(gidx_hbm.at[pl.ds(off, CHUNK)], gi[s], semi[s])
            pltpu.async_copy(sidx_hbm.at[pl.ds(off, CHUNK)], si[s], semi[s])

        def wait_idx(s):
            pltpu.make_async_copy(
                gidx_hbm.at[pl.ds(0, CHUNK)], gi[s], semi[s]).wait()
            pltpu.make_async_copy(
                sidx_hbm.at[pl.ds(0, CHUNK)], si[s], semi[s]).wait()

        def issue_gather(s):
            pltpu.async_copy(src_hbm.at[gi[s]], r[s], semg[s])

        def wait_gather(s):
            pltpu.make_async_copy(src_hbm.at[gi[s]], r[s], semg[s]).wait()

        def issue_scats(s):
            pltpu.async_copy(r[s], acc_sh.at[si[s]], sem_s, add=True)
            if with_counts:
                pltpu.async_copy(ones_v, cnt_sh.at[si[s]], sem_c, add=True)
                pltpu.async_copy(ones_v, dcnt_sh.at[gi[s]], sem_c, add=True)

        def drain_scats(s):
            pltpu.make_async_copy(r[s], acc_sh.at[si[s]], sem_s).wait()
            if with_counts:
                pltpu.make_async_copy(ones_v, cnt_sh.at[si[s]], sem_c).wait()
                pltpu.make_async_copy(ones_v, dcnt_sh.at[gi[s]], sem_c).wait()

        z16 = jnp.zeros((16,), jnp.float32)

        def zero_rows(i, _):
            for cblk in range(C // 16):
                r0[i, pl.ds(cblk * 16, 16)] = z16
            return 0

        lax.fori_loop(0, CHUNK, zero_rows, 0)

        def zinit(i, _):
            j = chunk_id(i)
            pltpu.sync_copy(r0, acc_sh.at[pl.ds(j * CHUNK, CHUNK)])
            return 0

        lax.fori_loop(0, nj, zinit, 0)

        if with_counts:
            o16 = jnp.ones((16,), jnp.float32)
            for k in range(CHUNK // 16):
                ones_v[pl.ds(k * 16, 16)] = o16
                stg_v[pl.ds(k * 16, 16)] = z16

            def zinit_cnt(i, _):
                j = chunk_id(i)
                pltpu.sync_copy(stg_v, cnt_sh.at[pl.ds(j * CHUNK, CHUNK)])
                pltpu.sync_copy(stg_v, dcnt_sh.at[pl.ds(j * CHUNK, CHUNK)])
                return 0

            lax.fori_loop(0, nj, zinit_cnt, 0)

        plsc.subcore_barrier()

        def chunk_body(j, s, s1, s2, drain=True, prefetch=True,
                       gather_next=True):
            wait_gather(s)
            issue_scats(s)
            if drain:
                drain_scats(s2)
            if prefetch:
                issue_idx(s2, j + 2)
            if gather_next:
                wait_idx(s1)
                issue_gather(s1)

        issue_idx(0, 0)
        issue_idx(1, 1)
        wait_idx(0)
        issue_gather(0)
        chunk_body(0, 0, 1, 2, drain=False)
        chunk_body(1, 1, 2, 0)
        chunk_body(2, 2, 0, 1)

        def steady(i, _):
            j = STEADY0 + 3 * i
            chunk_body(j, 0, 1, 2)
            chunk_body(j + 1, 1, 2, 0)
            chunk_body(j + 2, 2, 0, 1)
            return 0

        lax.fori_loop(0, STEADY_ITERS, steady, 0)

        chunk_body(N_CHUNKS - 2, 0, 1, 2, prefetch=False)
        chunk_body(N_CHUNKS - 1, 1, 2, 0, prefetch=False, gather_next=False)
        drain_scats(1)

        plsc.subcore_barrier()

        def wback(i, _):
            j = chunk_id(i)
            pltpu.sync_copy(acc_sh.at[pl.ds(j * CHUNK, CHUNK)], r0)
            pltpu.sync_copy(r0, out_hbm.at[cid, pl.ds(j * CHUNK, CHUNK)])
            return 0

        lax.fori_loop(0, nj, wback, 0)
        if with_counts:
            def wback_cnt(i, _):
                j = chunk_id(i)
                pltpu.sync_copy(cnt_sh.at[pl.ds(j * CHUNK, CHUNK)], stg_v)
                pltpu.sync_copy(
                    stg_v, cnt_hbm.at[pl.ds(cid * NUM_EDGES + j * CHUNK, CHUNK)])
                pltpu.sync_copy(dcnt_sh.at[pl.ds(j * CHUNK, CHUNK)], ones_v)
                pltpu.sync_copy(
                    ones_v, dcnt_hbm.at[pl.ds(cid * N_NODES + j * CHUNK, CHUNK)])
                return 0

            lax.fori_loop(0, nj, wback_cnt, 0)

    return functools.partial(
        pl.kernel, mesh=mesh, out_type=out_type, scratch_types=scratch
    )(body)


_stage_counts = _make_stage(with_counts=True)
_stage_plain = _make_stage(with_counts=False)


def _comb_body(part_ref, cnt_ref, o_ref):
    s = part_ref[0] + part_ref[1]
    c = (cnt_ref[0] + cnt_ref[1])[:, None]
    inv = jnp.where(c > 0.0, 1.0 / c, 0.0)
    o_ref[...] = s * inv


def _combine(part, cnt):
    return pl.pallas_call(
        _comb_body,
        out_shape=jax.ShapeDtypeStruct((SEG, C), jnp.float32),
    )(part, cnt)


def kernel(x, hyperedge_index, W):
    he = hyperedge_index.astype(jnp.int32)
    row = he[0]
    col = he[1]
    xw = _matmul(x, W)
    e_part, cnt, dcnt = _stage_counts(xw, row, col)
    edge_feat = _combine(e_part, cnt.reshape(NC, NUM_EDGES))
    (n_part,) = _stage_plain(edge_feat, col, row)
    out = _combine(n_part, dcnt.reshape(NC, N_NODES))
    return out
